# baseline (speedup 1.0000x reference)
import jax
import jax.numpy as jnp
from jax.experimental import pallas as pl

_Q = 8192
_K = 8192
_KNN = 11
_BQ = 128


def _sdf_block_kernel(q_ref, pt_ref, nt_ref, qq_ref, pp_ref, out_ref):
    q = q_ref[...]
    pt = pt_ref[...]
    nt = nt_ref[...]
    qq = qq_ref[...]
    pp = pp_ref[...]
    cross2 = jnp.dot(q, pt, preferred_element_type=jnp.float32)
    d2 = (qq - cross2) + pp
    s = ((q[:, 4:5] - pt[0:1, :]) * nt[0:1, :]
         + (q[:, 5:6] - pt[1:2, :]) * nt[1:2, :]) \
        + (q[:, 6:7] - pt[2:3, :]) * nt[2:3, :]
    sneg = s < 0.0

    inf = jnp.float32(jnp.inf)
    m1 = d2[:, 0:128]
    m2 = jnp.full((_BQ, 128), inf, dtype=jnp.float32)
    m3 = m2
    for c in range(1, _K // 128):
        v = d2[:, 128 * c:128 * (c + 1)]
        t1 = jnp.maximum(m1, v)
        m1 = jnp.minimum(m1, v)
        t2 = jnp.maximum(m2, t1)
        m2 = jnp.minimum(m2, t1)
        m3 = jnp.minimum(m3, t2)

    m0 = None
    t = None
    for i in range(_KNN):
        mm = jnp.minimum(jnp.minimum(m1, m2), m3)
        m = jnp.min(mm, axis=1, keepdims=True)
        if i == 0:
            m0 = m
        if i == _KNN - 1:
            t = m
        else:
            m1 = jnp.where(m1 == m, inf, m1)
            m2 = jnp.where(m2 == m, inf, m2)
            m3 = jnp.where(m3 == m, inf, m3)

    le = d2 <= t
    sv = jnp.where(sneg, jnp.int32(16385), jnp.int32(16384))
    packed = jnp.sum(jnp.where(le, sv, 0), axis=1, keepdims=True)
    cnt = packed >> 14
    votes = packed & 16383
    d0 = jnp.sqrt(jnp.maximum(m0, 0.0))
    out_ref[...] = jnp.where(votes * 2 > _KNN, -d0, d0)

    def _exact_path():
        iota = jax.lax.broadcasted_iota(jnp.int32, (_BQ, _K), 1)
        w2 = d2
        mask = jnp.zeros((_BQ, _K), dtype=jnp.bool_)
        for _ in range(_KNN):
            m2 = jnp.min(w2, axis=1, keepdims=True)
            cand = jnp.where(w2 == m2, iota, _K)
            fi = jnp.min(cand, axis=1, keepdims=True)
            hit = cand == fi
            mask = mask | hit
            w2 = jnp.where(hit, jnp.inf, w2)
        v2 = jnp.sum(jnp.where(mask & sneg, 1.0, 0.0), axis=1, keepdims=True)
        out_ref[...] = jnp.where(v2 > (_KNN * 0.5), -d0, d0)

    pl.when(jnp.any(cnt != _KNN))(_exact_path)


@jax.jit
def kernel(query_points, points, normals):
    qp = (jnp.zeros((_Q, 8), jnp.float32)
          .at[:, :3].set(2.0 * query_points)
          .at[:, 4:7].set(query_points))
    pt = jnp.zeros((8, _K), jnp.float32).at[:3, :].set(points.T)
    nt = jnp.zeros((8, _K), jnp.float32).at[:3, :].set(normals.T)
    qq = jnp.sum(query_points * query_points, axis=1)[:, None]
    pp = jnp.sum(points * points, axis=1)[None, :]
    out = pl.pallas_call(
        _sdf_block_kernel,
        grid=(_Q // _BQ,),
        in_specs=[
            pl.BlockSpec((_BQ, 8), lambda i: (i, 0)),
            pl.BlockSpec((8, _K), lambda i: (0, 0)),
            pl.BlockSpec((8, _K), lambda i: (0, 0)),
            pl.BlockSpec((_BQ, 1), lambda i: (i, 0)),
            pl.BlockSpec((1, _K), lambda i: (0, 0)),
        ],
        out_specs=pl.BlockSpec((_BQ, 1), lambda i: (i, 0)),
        out_shape=jax.ShapeDtypeStruct((_Q, 1), jnp.float32),
    )(qp, pt, nt, qq, pp)
    return out[:, 0]


# speedup vs baseline: 1.0181x; 1.0181x over previous
import jax
import jax.numpy as jnp
from jax.experimental import pallas as pl

_Q = 8192
_K = 8192
_KNN = 11
_BQ = 128


def _sdf_block_kernel(q_ref, pt_ref, nt_ref, qq_ref, pp_ref, out_ref):
    q = q_ref[...]
    pt = pt_ref[...]
    nt = nt_ref[...]
    qq = qq_ref[...]
    pp = pp_ref[...]
    cross2 = jnp.dot(q, pt, preferred_element_type=jnp.float32)
    d2 = (qq - cross2) + pp
    s = ((q[:, 4:5] - pt[0:1, :]) * nt[0:1, :]
         + (q[:, 5:6] - pt[1:2, :]) * nt[1:2, :]) \
        + (q[:, 6:7] - pt[2:3, :]) * nt[2:3, :]
    sneg = s < 0.0

    inf = jnp.float32(jnp.inf)
    m1 = d2[:, 0:128]
    m2 = jnp.full((_BQ, 128), inf, dtype=jnp.float32)
    m3 = m2
    for c in range(1, _K // 128):
        v = d2[:, 128 * c:128 * (c + 1)]
        t1 = jnp.maximum(m1, v)
        m1 = jnp.minimum(m1, v)
        t2 = jnp.maximum(m2, t1)
        m2 = jnp.minimum(m2, t1)
        m3 = jnp.minimum(m3, t2)

    m0 = None
    t = None
    for i in range(_KNN):
        mm = jnp.minimum(jnp.minimum(m1, m2), m3)
        m = jnp.min(mm, axis=1, keepdims=True)
        if i == 0:
            m0 = m
        if i == _KNN - 1:
            t = m
        else:
            m1 = jnp.where(m1 == m, inf, m1)
            m2 = jnp.where(m2 == m, inf, m2)
            m3 = jnp.where(m3 == m, inf, m3)

    le = d2 <= t
    sv = jnp.where(sneg, jnp.float32(1.0625), jnp.float32(1.0))
    packed = jnp.sum(jnp.where(le, sv, 0.0), axis=1, keepdims=True)
    cnt = jnp.floor(packed)
    votes = (packed - cnt) * 16.0
    d0 = jnp.sqrt(jnp.maximum(m0, 0.0))
    out_ref[...] = jnp.where(votes * 2.0 > _KNN, -d0, d0)

    def _exact_path():
        iota = jax.lax.broadcasted_iota(jnp.int32, (_BQ, _K), 1)
        w2 = d2
        mask = jnp.zeros((_BQ, _K), dtype=jnp.bool_)
        for _ in range(_KNN):
            m2 = jnp.min(w2, axis=1, keepdims=True)
            cand = jnp.where(w2 == m2, iota, _K)
            fi = jnp.min(cand, axis=1, keepdims=True)
            hit = cand == fi
            mask = mask | hit
            w2 = jnp.where(hit, jnp.inf, w2)
        v2 = jnp.sum(jnp.where(mask & sneg, 1.0, 0.0), axis=1, keepdims=True)
        out_ref[...] = jnp.where(v2 > (_KNN * 0.5), -d0, d0)

    pl.when(jnp.any(cnt != _KNN))(_exact_path)


@jax.jit
def kernel(query_points, points, normals):
    qp = (jnp.zeros((_Q, 8), jnp.float32)
          .at[:, :3].set(2.0 * query_points)
          .at[:, 4:7].set(query_points))
    pt = jnp.zeros((8, _K), jnp.float32).at[:3, :].set(points.T)
    nt = jnp.zeros((8, _K), jnp.float32).at[:3, :].set(normals.T)
    qq = jnp.sum(query_points * query_points, axis=1)[:, None]
    pp = jnp.sum(points * points, axis=1)[None, :]
    out = pl.pallas_call(
        _sdf_block_kernel,
        grid=(_Q // _BQ,),
        in_specs=[
            pl.BlockSpec((_BQ, 8), lambda i: (i, 0)),
            pl.BlockSpec((8, _K), lambda i: (0, 0)),
            pl.BlockSpec((8, _K), lambda i: (0, 0)),
            pl.BlockSpec((_BQ, 1), lambda i: (i, 0)),
            pl.BlockSpec((1, _K), lambda i: (0, 0)),
        ],
        out_specs=pl.BlockSpec((_BQ, 1), lambda i: (i, 0)),
        out_shape=jax.ShapeDtypeStruct((_Q, 1), jnp.float32),
    )(qp, pt, nt, qq, pp)
    return out[:, 0]


# speedup vs baseline: 1.0619x; 1.0431x over previous
import jax
import jax.numpy as jnp
from jax.experimental import pallas as pl

_Q = 8192
_K = 8192
_KNN = 11
_BQ = 128


def _sdf_block_kernel(q_ref, pt_ref, nt_ref, qq_ref, pp_ref, out_ref):
    q = q_ref[...]
    pt = pt_ref[...]
    nt = nt_ref[...]
    qq = qq_ref[...]
    pp = pp_ref[...]
    cross = jnp.dot(q, pt, preferred_element_type=jnp.float32)
    d2 = (qq - 2.0 * cross) + pp
    s = ((q[:, 0:1] - pt[0:1, :]) * nt[0:1, :]
         + (q[:, 1:2] - pt[1:2, :]) * nt[1:2, :]) \
        + (q[:, 2:3] - pt[2:3, :]) * nt[2:3, :]
    sneg = s < 0.0

    inf = jnp.float32(jnp.inf)
    m1 = d2[:, 0:128]
    m2 = jnp.full((_BQ, 128), inf, dtype=jnp.float32)
    m3 = m2
    for c in range(1, _K // 128):
        v = d2[:, 128 * c:128 * (c + 1)]
        t1 = jnp.maximum(m1, v)
        m1 = jnp.minimum(m1, v)
        t2 = jnp.maximum(m2, t1)
        m2 = jnp.minimum(m2, t1)
        m3 = jnp.minimum(m3, t2)

    m0 = None
    t = None
    for i in range(_KNN):
        mm = jnp.minimum(jnp.minimum(m1, m2), m3)
        m = jnp.min(mm, axis=1, keepdims=True)
        if i == 0:
            m0 = m
        if i == _KNN - 1:
            t = m
        else:
            m1 = jnp.where(m1 == m, inf, m1)
            m2 = jnp.where(m2 == m, inf, m2)
            m3 = jnp.where(m3 == m, inf, m3)

    le = d2 <= t
    ones = jnp.where(le, 1.0, 0.0)
    cnt = jnp.sum(ones, axis=1, keepdims=True)
    votes = jnp.sum(jnp.where(le & sneg, 1.0, 0.0), axis=1, keepdims=True)
    d0 = jnp.sqrt(jnp.maximum(m0, 0.0))
    out_ref[...] = jnp.where(votes > (_KNN * 0.5), -d0, d0)

    def _exact_path():
        iota = jax.lax.broadcasted_iota(jnp.int32, (_BQ, _K), 1)
        w2 = d2
        mask = jnp.zeros((_BQ, _K), dtype=jnp.bool_)
        for _ in range(_KNN):
            m2 = jnp.min(w2, axis=1, keepdims=True)
            cand = jnp.where(w2 == m2, iota, _K)
            fi = jnp.min(cand, axis=1, keepdims=True)
            hit = cand == fi
            mask = mask | hit
            w2 = jnp.where(hit, jnp.inf, w2)
        v2 = jnp.sum(jnp.where(mask & sneg, 1.0, 0.0), axis=1, keepdims=True)
        out_ref[...] = jnp.where(v2 > (_KNN * 0.5), -d0, d0)

    pl.when(jnp.any(cnt != float(_KNN)))(_exact_path)


@jax.jit
def kernel(query_points, points, normals):
    qp = jnp.zeros((_Q, 8), jnp.float32).at[:, :3].set(query_points)
    pt = jnp.zeros((8, _K), jnp.float32).at[:3, :].set(points.T)
    nt = jnp.zeros((8, _K), jnp.float32).at[:3, :].set(normals.T)
    qq = jnp.sum(query_points * query_points, axis=1)[:, None]
    pp = jnp.sum(points * points, axis=1)[None, :]
    out = pl.pallas_call(
        _sdf_block_kernel,
        grid=(_Q // _BQ,),
        in_specs=[
            pl.BlockSpec((_BQ, 8), lambda i: (i, 0)),
            pl.BlockSpec((8, _K), lambda i: (0, 0)),
            pl.BlockSpec((8, _K), lambda i: (0, 0)),
            pl.BlockSpec((_BQ, 1), lambda i: (i, 0)),
            pl.BlockSpec((1, _K), lambda i: (0, 0)),
        ],
        out_specs=pl.BlockSpec((_BQ, 1), lambda i: (i, 0)),
        out_shape=jax.ShapeDtypeStruct((_Q, 1), jnp.float32),
    )(qp, pt, nt, qq, pp)
    return out[:, 0]


# speedup vs baseline: 1.1070x; 1.0424x over previous
import jax
import jax.numpy as jnp
from jax.experimental import pallas as pl

_Q = 8192
_K = 8192
_KNN = 11
_BQ = 128


def _sdf_block_kernel(q_ref, pt_ref, nt_ref, qq_ref, pp_ref, out_ref):
    q = q_ref[...]
    pt = pt_ref[...]
    nt = nt_ref[...]
    qq = qq_ref[...]
    pp = pp_ref[...]
    cross = jnp.dot(q, pt, preferred_element_type=jnp.float32)
    d2 = (qq - 2.0 * cross) + pp
    s = ((q[:, 0:1] - pt[0:1, :]) * nt[0:1, :]
         + (q[:, 1:2] - pt[1:2, :]) * nt[1:2, :]) \
        + (q[:, 2:3] - pt[2:3, :]) * nt[2:3, :]
    sneg = s < 0.0

    inf = jnp.float32(jnp.inf)
    m1 = d2[:, 0:128]
    m2 = jnp.full((_BQ, 128), inf, dtype=jnp.float32)
    m3 = m2
    for c in range(1, _K // 128):
        v = d2[:, 128 * c:128 * (c + 1)]
        t1 = jnp.maximum(m1, v)
        m1 = jnp.minimum(m1, v)
        t2 = jnp.maximum(m2, t1)
        m2 = jnp.minimum(m2, t1)
        m3 = jnp.minimum(m3, t2)

    m0 = None
    t = None
    for i in range(_KNN):
        mm = jnp.minimum(jnp.minimum(m1, m2), m3)
        m = jnp.min(mm, axis=1, keepdims=True)
        if i == 0:
            m0 = m
        if i == _KNN - 1:
            t = m
        else:
            m1 = jnp.where(m1 == m, inf, m1)
            m2 = jnp.where(m2 == m, inf, m2)
            m3 = jnp.where(m3 == m, inf, m3)

    le = d2 <= t
    sv = jnp.where(sneg, jnp.float32(1.0625), jnp.float32(1.0))
    packed = jnp.sum(jnp.where(le, sv, 0.0), axis=1, keepdims=True)
    cnt = jnp.floor(packed)
    votes = (packed - cnt) * 16.0
    d0 = jnp.sqrt(jnp.maximum(m0, 0.0))
    out_ref[...] = jnp.where(votes > (_KNN * 0.5), -d0, d0)

    def _exact_path():
        iota = jax.lax.broadcasted_iota(jnp.int32, (_BQ, _K), 1)
        w2 = d2
        mask = jnp.zeros((_BQ, _K), dtype=jnp.bool_)
        for _ in range(_KNN):
            m2 = jnp.min(w2, axis=1, keepdims=True)
            cand = jnp.where(w2 == m2, iota, _K)
            fi = jnp.min(cand, axis=1, keepdims=True)
            hit = cand == fi
            mask = mask | hit
            w2 = jnp.where(hit, jnp.inf, w2)
        v2 = jnp.sum(jnp.where(mask & sneg, 1.0, 0.0), axis=1, keepdims=True)
        out_ref[...] = jnp.where(v2 > (_KNN * 0.5), -d0, d0)

    pl.when(jnp.any(cnt != float(_KNN)))(_exact_path)


@jax.jit
def kernel(query_points, points, normals):
    qp = jnp.zeros((_Q, 8), jnp.float32).at[:, :3].set(query_points)
    pt = jnp.zeros((8, _K), jnp.float32).at[:3, :].set(points.T)
    nt = jnp.zeros((8, _K), jnp.float32).at[:3, :].set(normals.T)
    qq = jnp.sum(query_points * query_points, axis=1)[:, None]
    pp = jnp.sum(points * points, axis=1)[None, :]
    out = pl.pallas_call(
        _sdf_block_kernel,
        grid=(_Q // _BQ,),
        in_specs=[
            pl.BlockSpec((_BQ, 8), lambda i: (i, 0)),
            pl.BlockSpec((8, _K), lambda i: (0, 0)),
            pl.BlockSpec((8, _K), lambda i: (0, 0)),
            pl.BlockSpec((_BQ, 1), lambda i: (i, 0)),
            pl.BlockSpec((1, _K), lambda i: (0, 0)),
        ],
        out_specs=pl.BlockSpec((_BQ, 1), lambda i: (i, 0)),
        out_shape=jax.ShapeDtypeStruct((_Q, 1), jnp.float32),
    )(qp, pt, nt, qq, pp)
    return out[:, 0]
